# baseline (speedup 1.0000x reference)
import functools

import jax
import jax.numpy as jnp
from jax import lax
from jax.experimental import pallas as pl
from jax.experimental.pallas import tpu as pltpu
from jax.experimental.pallas import tpu_sc as plsc

_SC_CORES = 2
_SC_SUBCORES = 16
_SC_WORKERS = _SC_CORES * _SC_SUBCORES


def _sc_gather(emb, idx_padded, rows_per_worker):
    n_pad = idx_padded.shape[0]
    d = emb.shape[1]
    mesh = plsc.VectorSubcoreMesh(core_axis_name="c", subcore_axis_name="s")

    @functools.partial(
        pl.kernel,
        mesh=mesh,
        compiler_params=pltpu.CompilerParams(use_tc_tiling_on_sc=False),
        out_type=jax.ShapeDtypeStruct((n_pad, d), jnp.float32),
        scratch_types=[
            pltpu.VMEM((rows_per_worker,), jnp.int32),
            pltpu.VMEM((rows_per_worker, d), jnp.float32),
            pltpu.SemaphoreType.DMA,
        ],
    )
    def k(table_hbm, idx_hbm, out_hbm, idx_v, rows_v, sem):
        wid = lax.axis_index("s") * _SC_CORES + lax.axis_index("c")
        base = wid * rows_per_worker
        pltpu.sync_copy(idx_hbm.at[pl.ds(base, rows_per_worker)], idx_v)
        pltpu.async_copy(table_hbm.at[idx_v], rows_v, sem).wait()
        pltpu.sync_copy(rows_v, out_hbm.at[pl.ds(base, rows_per_worker)])

    return k(emb, idx_padded)


def _hid_body(e_ref, w1_ref, b1_ref, hid_ref):
    k = pl.program_id(0)

    @pl.when(k == 0)
    def _():
        hid_ref[...] = b1_ref[...]

    hid_ref[...] += lax.dot_general(
        e_ref[...], w1_ref[...], (((1,), (1,)), ((), ())),
        preferred_element_type=jnp.float32)

    @pl.when(k == pl.num_programs(0) - 1)
    def _():
        hid_ref[...] = jnp.maximum(hid_ref[...], 0.0)


def _hid_matmul(emb_flat, w1, b1_row, k_block):
    in1 = emb_flat.shape[1]
    hidden = w1.shape[0]
    grid = in1 // k_block
    return pl.pallas_call(
        _hid_body,
        grid=(grid,),
        in_specs=[
            pl.BlockSpec((1, k_block), lambda k: (0, k)),
            pl.BlockSpec((hidden, k_block), lambda k: (0, k)),
            pl.BlockSpec((1, hidden), lambda k: (0, 0)),
        ],
        out_specs=pl.BlockSpec((1, hidden), lambda k: (0, 0)),
        out_shape=jax.ShapeDtypeStruct((1, hidden), jnp.float32),
    )(emb_flat, w1, b1_row)


def _out_body(vocab, hid_ref, w2_ref, b2_ref, out_ref, lse_ref, m_ref, s_ref):
    i = pl.program_id(0)
    o = lax.dot_general(
        hid_ref[...], w2_ref[...], (((1,), (1,)), ((), ())),
        preferred_element_type=jnp.float32) + b2_ref[...]
    out_ref[...] = o
    v_block = o.shape[1]
    col = i * v_block + lax.broadcasted_iota(jnp.int32, o.shape, 1)
    o = jnp.where(col < vocab, o, -jnp.inf)
    t = jnp.max(o, keepdims=True)

    @pl.when(i == 0)
    def _():
        m_ref[...] = t
        s_ref[...] = jnp.sum(jnp.exp(o - t), keepdims=True)

    @pl.when(i > 0)
    def _():
        m_old = m_ref[...]
        m_new = jnp.maximum(m_old, t)
        m_ref[...] = m_new
        s_ref[...] = (s_ref[...] * jnp.exp(m_old - m_new)
                      + jnp.sum(jnp.exp(o - m_new), keepdims=True))

    @pl.when(i == pl.num_programs(0) - 1)
    def _():
        lse_ref[...] = m_ref[...] + jnp.log(s_ref[...])


def _out_matmul(hid, w2, b2_row, v_block):
    vocab = w2.shape[0]
    hidden = w2.shape[1]
    grid = -(-vocab // v_block)
    return pl.pallas_call(
        functools.partial(_out_body, vocab),
        grid=(grid,),
        in_specs=[
            pl.BlockSpec((1, hidden), lambda i: (0, 0)),
            pl.BlockSpec((v_block, hidden), lambda i: (i, 0)),
            pl.BlockSpec((1, v_block), lambda i: (0, i)),
        ],
        out_specs=[
            pl.BlockSpec((1, v_block), lambda i: (0, i)),
            pl.BlockSpec((1, 1), lambda i: (0, 0)),
        ],
        out_shape=[
            jax.ShapeDtypeStruct((1, vocab), jnp.float32),
            jax.ShapeDtypeStruct((1, 1), jnp.float32),
        ],
        scratch_shapes=[
            pltpu.VMEM((1, 1), jnp.float32),
            pltpu.VMEM((1, 1), jnp.float32),
        ],
    )(hid, w2, b2_row)


def _sub_body(out_ref, lse_ref, lp_ref):
    lp_ref[...] = out_ref[...] - lse_ref[...]


def _log_probs(out, lse, v_block):
    vocab = out.shape[1]
    grid = -(-vocab // v_block)
    return pl.pallas_call(
        _sub_body,
        grid=(grid,),
        in_specs=[
            pl.BlockSpec((1, v_block), lambda i: (0, i)),
            pl.BlockSpec((1, 1), lambda i: (0, 0)),
        ],
        out_specs=pl.BlockSpec((1, v_block), lambda i: (0, i)),
        out_shape=jax.ShapeDtypeStruct((1, vocab), jnp.float32),
    )(out, lse)


def kernel(inputs, emb, W1, b1, W2, b2):
    n_ctx = inputs.shape[0]
    rows_per_worker = -(-n_ctx // (8 * _SC_WORKERS)) * 8
    n_pad = rows_per_worker * _SC_WORKERS
    idx = jnp.pad(inputs.astype(jnp.int32), (0, n_pad - n_ctx))
    rows = _sc_gather(emb, idx, rows_per_worker)
    emb_flat = rows[:n_ctx].reshape(1, -1)
    hid = _hid_matmul(emb_flat, W1, b1.reshape(1, -1), k_block=1280)
    out, lse = _out_matmul(hid, W2, b2.reshape(1, -1), v_block=2048)
    return _log_probs(out, lse, v_block=2048)


# speedup vs baseline: 1.1425x; 1.1425x over previous
import functools

import jax
import jax.numpy as jnp
from jax import lax
from jax.experimental import pallas as pl
from jax.experimental.pallas import tpu as pltpu
from jax.experimental.pallas import tpu_sc as plsc

_SC_CORES = 2
_SC_SUBCORES = 16
_SC_WORKERS = _SC_CORES * _SC_SUBCORES


def _sc_gather(emb, idx_padded, rows_per_worker):
    n_pad = idx_padded.shape[0]
    d = emb.shape[1]
    per_core = n_pad // _SC_CORES
    mesh = plsc.ScalarSubcoreMesh(axis_name="c", num_cores=_SC_CORES)

    @functools.partial(
        pl.kernel,
        mesh=mesh,
        out_type=jax.ShapeDtypeStruct((n_pad, d), jnp.float32),
        scratch_types=[
            pltpu.SMEM((per_core,), jnp.int32),
            pltpu.SemaphoreType.DMA,
        ],
    )
    def k(table_hbm, idx_hbm, out_hbm, idx_s, sem):
        cid = lax.axis_index("c")
        base = cid * per_core
        pltpu.sync_copy(idx_hbm.at[pl.ds(base, per_core)], idx_s)
        copies = [
            pltpu.async_copy(table_hbm.at[idx_s[r]], out_hbm.at[base + r], sem)
            for r in range(per_core)
        ]
        for c in copies:
            c.wait()

    return k(emb, idx_padded)


def _hid_body(e_ref, w1_ref, b1_ref, hid_ref):
    k = pl.program_id(0)

    @pl.when(k == 0)
    def _():
        hid_ref[...] = b1_ref[...]

    hid_ref[...] += lax.dot_general(
        e_ref[...], w1_ref[...], (((1,), (1,)), ((), ())),
        preferred_element_type=jnp.float32)

    @pl.when(k == pl.num_programs(0) - 1)
    def _():
        hid_ref[...] = jnp.maximum(hid_ref[...], 0.0)


def _hid_matmul(emb_flat, w1, b1_row, k_block):
    in1 = emb_flat.shape[1]
    hidden = w1.shape[0]
    grid = in1 // k_block
    return pl.pallas_call(
        _hid_body,
        grid=(grid,),
        in_specs=[
            pl.BlockSpec((1, k_block), lambda k: (0, k)),
            pl.BlockSpec((hidden, k_block), lambda k: (0, k)),
            pl.BlockSpec((1, hidden), lambda k: (0, 0)),
        ],
        out_specs=pl.BlockSpec((1, hidden), lambda k: (0, 0)),
        out_shape=jax.ShapeDtypeStruct((1, hidden), jnp.float32),
    )(emb_flat, w1, b1_row)


def _out_body(vocab, hid_ref, w2_ref, b2_ref, out_ref, lse_ref, m_ref, s_ref):
    i = pl.program_id(0)
    o = lax.dot_general(
        hid_ref[...], w2_ref[...], (((1,), (1,)), ((), ())),
        preferred_element_type=jnp.float32) + b2_ref[...]
    out_ref[...] = o
    v_block = o.shape[1]
    col = i * v_block + lax.broadcasted_iota(jnp.int32, o.shape, 1)
    o = jnp.where(col < vocab, o, -jnp.inf)
    t = jnp.max(o, keepdims=True)

    @pl.when(i == 0)
    def _():
        m_ref[...] = t
        s_ref[...] = jnp.sum(jnp.exp(o - t), keepdims=True)

    @pl.when(i > 0)
    def _():
        m_old = m_ref[...]
        m_new = jnp.maximum(m_old, t)
        m_ref[...] = m_new
        s_ref[...] = (s_ref[...] * jnp.exp(m_old - m_new)
                      + jnp.sum(jnp.exp(o - m_new), keepdims=True))

    @pl.when(i == pl.num_programs(0) - 1)
    def _():
        lse_ref[...] = m_ref[...] + jnp.log(s_ref[...])


def _out_matmul(hid, w2, b2_row, v_block):
    vocab = w2.shape[0]
    hidden = w2.shape[1]
    grid = -(-vocab // v_block)
    return pl.pallas_call(
        functools.partial(_out_body, vocab),
        grid=(grid,),
        in_specs=[
            pl.BlockSpec((1, hidden), lambda i: (0, 0)),
            pl.BlockSpec((v_block, hidden), lambda i: (i, 0)),
            pl.BlockSpec((1, v_block), lambda i: (0, i)),
        ],
        out_specs=[
            pl.BlockSpec((1, v_block), lambda i: (0, i)),
            pl.BlockSpec((1, 1), lambda i: (0, 0)),
        ],
        out_shape=[
            jax.ShapeDtypeStruct((1, vocab), jnp.float32),
            jax.ShapeDtypeStruct((1, 1), jnp.float32),
        ],
        scratch_shapes=[
            pltpu.VMEM((1, 1), jnp.float32),
            pltpu.VMEM((1, 1), jnp.float32),
        ],
    )(hid, w2, b2_row)


def _sub_body(out_ref, lse_ref, lp_ref):
    lp_ref[...] = out_ref[...] - lse_ref[...]


def _log_probs(out, lse, v_block):
    vocab = out.shape[1]
    grid = -(-vocab // v_block)
    return pl.pallas_call(
        _sub_body,
        grid=(grid,),
        in_specs=[
            pl.BlockSpec((1, v_block), lambda i: (0, i)),
            pl.BlockSpec((1, 1), lambda i: (0, 0)),
        ],
        out_specs=pl.BlockSpec((1, v_block), lambda i: (0, i)),
        out_shape=jax.ShapeDtypeStruct((1, vocab), jnp.float32),
    )(out, lse)


def kernel(inputs, emb, W1, b1, W2, b2):
    n_ctx = inputs.shape[0]
    rows_per_worker = -(-n_ctx // (8 * _SC_WORKERS)) * 8
    n_pad = rows_per_worker * _SC_WORKERS
    idx = jnp.pad(inputs.astype(jnp.int32), (0, n_pad - n_ctx))
    rows = _sc_gather(emb, idx, rows_per_worker)
    emb_flat = rows[:n_ctx].reshape(1, -1)
    hid = _hid_matmul(emb_flat, W1, b1.reshape(1, -1), k_block=1280)
    out, lse = _out_matmul(hid, W2, b2.reshape(1, -1), v_block=2048)
    return _log_probs(out, lse, v_block=2048)


# speedup vs baseline: 1.6844x; 1.4744x over previous
import functools

import jax
import jax.numpy as jnp
from jax import lax
from jax.experimental import pallas as pl
from jax.experimental.pallas import tpu as pltpu


def _gather_body(idx_ref, emb_ref, rows_ref, sem):
    n = rows_ref.shape[0]
    copies = [
        pltpu.make_async_copy(
            emb_ref.at[pl.ds(idx_ref[r], 1)], rows_ref.at[pl.ds(r, 1)], sem)
        for r in range(n)
    ]
    for c in copies:
        c.start()
    for c in copies:
        c.wait()


def _tc_gather(emb, idx):
    n = idx.shape[0]
    d = emb.shape[1]
    return pl.pallas_call(
        _gather_body,
        in_specs=[
            pl.BlockSpec(memory_space=pltpu.SMEM),
            pl.BlockSpec(memory_space=pl.ANY),
        ],
        out_specs=pl.BlockSpec(memory_space=pl.ANY),
        out_shape=jax.ShapeDtypeStruct((n, d), jnp.float32),
        scratch_shapes=[pltpu.SemaphoreType.DMA],
    )(idx, emb)


def _mlp_body(nk, nv, v_block, vocab,
              e_ref, w1_ref, b1_ref, w2_ref, b2_ref, lp_ref,
              hid_s, m_s, s_s):
    i = pl.program_id(0)

    @pl.when(i == 0)
    def _():
        hid_s[...] = b1_ref[...]

    @pl.when(i < nk)
    def _():
        hid_s[...] += lax.dot_general(
            e_ref[...], w1_ref[...], (((1,), (1,)), ((), ())),
            preferred_element_type=jnp.float32)

        @pl.when(i == nk - 1)
        def _():
            hid_s[...] = jnp.maximum(hid_s[...], 0.0)

    @pl.when((i >= nk) & (i < nk + nv))
    def _():
        j = i - nk
        o = lax.dot_general(
            hid_s[...], w2_ref[...], (((1,), (1,)), ((), ())),
            preferred_element_type=jnp.float32) + b2_ref[...]
        lp_ref[0, pl.ds(j * v_block, v_block)] = o[0]
        col = j * v_block + lax.broadcasted_iota(jnp.int32, o.shape, 1)
        om = jnp.where(col < vocab, o, -jnp.inf)
        t = jnp.max(om, keepdims=True)

        @pl.when(j == 0)
        def _():
            m_s[...] = t
            s_s[...] = jnp.sum(jnp.exp(om - t), keepdims=True)

        @pl.when(j > 0)
        def _():
            m_old = m_s[...]
            m_new = jnp.maximum(m_old, t)
            m_s[...] = m_new
            s_s[...] = (s_s[...] * jnp.exp(m_old - m_new)
                        + jnp.sum(jnp.exp(om - m_new), keepdims=True))

    @pl.when(i == nk + nv)
    def _():
        lp_ref[...] = lp_ref[...] - (m_s[...] + jnp.log(s_s[...]))


def _mlp(emb_flat, w1, b1_row, w2, b2_row, k_block, v_block):
    in1 = emb_flat.shape[1]
    hidden = w1.shape[0]
    vocab = w2.shape[0]
    nk = in1 // k_block
    nv = -(-vocab // v_block)
    v_pad = nv * v_block
    grid = nk + nv + 1

    def w1_idx(i):
        return (0, jnp.minimum(i, nk - 1))

    def w2_idx(i):
        return (jnp.clip(i - nk, 0, nv - 1), 0)

    def b2_idx(i):
        return (0, jnp.clip(i - nk, 0, nv - 1))

    lp = pl.pallas_call(
        functools.partial(_mlp_body, nk, nv, v_block, vocab),
        grid=(grid,),
        in_specs=[
            pl.BlockSpec((1, k_block), w1_idx),
            pl.BlockSpec((hidden, k_block), w1_idx),
            pl.BlockSpec((1, hidden), lambda i: (0, 0)),
            pl.BlockSpec((v_block, hidden), w2_idx),
            pl.BlockSpec((1, v_block), b2_idx),
        ],
        out_specs=pl.BlockSpec((1, v_pad), lambda i: (0, 0)),
        out_shape=jax.ShapeDtypeStruct((1, v_pad), jnp.float32),
        scratch_shapes=[
            pltpu.VMEM((1, hidden), jnp.float32),
            pltpu.VMEM((1, 1), jnp.float32),
            pltpu.VMEM((1, 1), jnp.float32),
        ],
    )(emb_flat, w1, b1_row, w2, b2_row)
    return lp[:, :vocab]


def kernel(inputs, emb, W1, b1, W2, b2):
    idx = inputs.astype(jnp.int32)
    rows = _tc_gather(emb, idx)
    emb_flat = rows.reshape(1, -1)
    return _mlp(emb_flat, W1, b1.reshape(1, -1), W2, b2.reshape(1, -1),
                k_block=2560, v_block=4096)


# speedup vs baseline: 2.4618x; 1.4615x over previous
import functools

import jax
import jax.numpy as jnp
from jax import lax
from jax.experimental import pallas as pl
from jax.experimental.pallas import tpu as pltpu


def _body(nk, nv, k_block, v_block, vocab, n_ctx, d,
          idx_ref, embT_ref, w1_ref, b1_ref, w2_ref, b2_ref, lp_ref,
          bufs, eflat, hid_s, m_s, s_s, sem):
    i = pl.program_id(0)

    @pl.when(i == 0)
    def _gather():
        copies = []
        for r in range(n_ctx):
            base = pl.multiple_of((idx_ref[r] // 128) * 128, 128)
            copies.append(pltpu.make_async_copy(
                embT_ref.at[:, pl.ds(base, 128)],
                bufs.at[:, pl.ds(r * 128, 128)], sem))
        for c in copies:
            c.start()
        for c in copies:
            c.wait()
        lane = lax.broadcasted_iota(jnp.int32, (1, 128), 1)
        for p in range(n_ctx // 2):
            segs = []
            for r in (2 * p, 2 * p + 1):
                off = idx_ref[r] - (idx_ref[r] // 128) * 128
                oh = (lane == off).astype(jnp.float32)
                segs.append(lax.dot_general(
                    oh, bufs[:, pl.ds(r * 128, 128)],
                    (((1,), (1,)), ((), ())),
                    preferred_element_type=jnp.float32))
            eflat[0, pl.ds(p * 2 * d, 2 * d)] = (
                jnp.concatenate(segs, axis=1)[0])

    @pl.when((i >= 1) & (i < 1 + nk))
    def _w1():
        k = i - 1

        @pl.when(k == 0)
        def _():
            hid_s[...] = b1_ref[...]

        hid_s[...] += lax.dot_general(
            eflat[:, pl.ds(k * k_block, k_block)], w1_ref[...],
            (((1,), (1,)), ((), ())),
            preferred_element_type=jnp.float32)

        @pl.when(k == nk - 1)
        def _():
            hid_s[...] = jnp.maximum(hid_s[...], 0.0)

    @pl.when((i >= 1 + nk) & (i < 1 + nk + nv))
    def _w2():
        j = i - 1 - nk
        o = lax.dot_general(
            hid_s[...], w2_ref[...], (((1,), (1,)), ((), ())),
            preferred_element_type=jnp.float32) + b2_ref[...]
        lp_ref[0, pl.ds(j * v_block, v_block)] = o[0]
        col = j * v_block + lax.broadcasted_iota(jnp.int32, o.shape, 1)
        om = jnp.where(col < vocab, o, -jnp.inf)
        t = jnp.max(om, keepdims=True)

        @pl.when(j == 0)
        def _():
            m_s[...] = t
            s_s[...] = jnp.sum(jnp.exp(om - t), keepdims=True)

        @pl.when(j > 0)
        def _():
            m_old = m_s[...]
            m_new = jnp.maximum(m_old, t)
            m_s[...] = m_new
            s_s[...] = (s_s[...] * jnp.exp(m_old - m_new)
                        + jnp.sum(jnp.exp(om - m_new), keepdims=True))

    @pl.when(i == 1 + nk + nv)
    def _finish():
        lp_ref[...] = lp_ref[...] - (m_s[...] + jnp.log(s_s[...]))


def kernel(inputs, emb, W1, b1, W2, b2):
    idx = inputs.astype(jnp.int32)
    embT = emb.T
    n_ctx = idx.shape[0]
    d = embT.shape[0]
    vocab, hidden = W2.shape
    in1 = W1.shape[1]
    k_block = 2560
    v_block = 4096
    nk = in1 // k_block
    nv = -(-vocab // v_block)
    v_pad = nv * v_block
    grid = 1 + nk + nv + 1

    def w1_idx(i):
        return (0, jnp.clip(i - 1, 0, nk - 1))

    def w2_idx(i):
        return (jnp.clip(i - 1 - nk, 0, nv - 1), 0)

    def b2_idx(i):
        return (0, jnp.clip(i - 1 - nk, 0, nv - 1))

    lp = pl.pallas_call(
        functools.partial(_body, nk, nv, k_block, v_block, vocab, n_ctx, d),
        grid=(grid,),
        in_specs=[
            pl.BlockSpec(memory_space=pltpu.SMEM),
            pl.BlockSpec(memory_space=pl.ANY),
            pl.BlockSpec((hidden, k_block), w1_idx),
            pl.BlockSpec((1, hidden), lambda i: (0, 0)),
            pl.BlockSpec((v_block, hidden), w2_idx),
            pl.BlockSpec((1, v_block), b2_idx),
        ],
        out_specs=pl.BlockSpec((1, v_pad), lambda i: (0, 0)),
        out_shape=jax.ShapeDtypeStruct((1, v_pad), jnp.float32),
        scratch_shapes=[
            pltpu.VMEM((d, n_ctx * 128), jnp.float32),
            pltpu.VMEM((1, n_ctx * d), jnp.float32),
            pltpu.VMEM((1, hidden), jnp.float32),
            pltpu.VMEM((1, 1), jnp.float32),
            pltpu.VMEM((1, 1), jnp.float32),
            pltpu.SemaphoreType.DMA,
        ],
    )(idx, embT, W1, b1.reshape(1, -1), W2, b2.reshape(1, -1))
    return lp[:, :vocab]
